# Initial kernel scaffold; baseline (speedup 1.0000x reference)
import jax
import jax.numpy as jnp
from jax.experimental import pallas as pl


def kernel(node_feat, edge_feat, edge_index, params):
    raise NotImplementedError("write your pallas kernel here")


# speedup vs baseline: 2.0279x; 2.0279x over previous
import functools

import jax
import jax.numpy as jnp
from jax import lax
from jax.experimental import pallas as pl
from jax.experimental.pallas import tpu as pltpu
from jax.experimental.pallas import tpu_sc as plsc

N = 10000
E = 320000
L = 128
NC = 2
NS = 16
NW = NC * NS
EW = E // NW
CH = 80
NCH = EW // CH
RT = 640
NPAD = NS * RT

BE = 2000
BN = 2000

_f32 = jnp.float32


def _mesh():
    return plsc.VectorSubcoreMesh(
        core_axis_name="c", subcore_axis_name="s", num_cores=NC, num_subcores=NS
    )


def _sc_gather_body(xa_hbm, xb_hbm, src_hbm, dst_hbm, g_hbm,
                    ia, ib, ra, rb, sema, semb):
    cid = lax.axis_index("c")
    sid = lax.axis_index("s")
    wid = sid * NC + cid
    base = wid * EW

    @pl.loop(0, NCH)
    def _chunk(j):
        off = base + j * CH
        pltpu.sync_copy(src_hbm.at[pl.ds(off, CH)], ia)
        pltpu.sync_copy(dst_hbm.at[pl.ds(off, CH)], ib)
        da = pltpu.async_copy(xa_hbm.at[ia], ra, sema)
        db = pltpu.async_copy(xb_hbm.at[ib], rb, semb)
        da.wait()
        db.wait()

        @pl.loop(0, CH)
        def _row(r):
            for k in range(L // 16):
                s = pl.ds(k * 16, 16)
                ra[r, s] = ra[r, s] + rb[r, s]

        pltpu.sync_copy(ra, g_hbm.at[pl.ds(off, CH)])


def _sc_gather(xa, xb, src, dst):
    fn = pl.kernel(
        _sc_gather_body,
        out_type=jax.ShapeDtypeStruct((E, L), _f32),
        mesh=_mesh(),
        scratch_types=[
            pltpu.VMEM((CH,), jnp.int32),
            pltpu.VMEM((CH,), jnp.int32),
            pltpu.VMEM((CH, L), _f32),
            pltpu.VMEM((CH, L), _f32),
            pltpu.SemaphoreType.DMA,
            pltpu.SemaphoreType.DMA,
        ],
    )
    return fn(xa, xb, src, dst)


def _sc_scatter_body(vals_hbm, idx_hbm, z128_hbm, out0_hbm, out1_hbm,
                     idx_v, rows_v, acc):
    cid = lax.axis_index("c")
    sid = lax.axis_index("s")
    wid = sid * NC + cid
    base = wid * EW
    rbase = sid * RT

    pltpu.sync_copy(z128_hbm, rows_v)

    @pl.loop(0, RT // CH)
    def _z(k):
        pltpu.sync_copy(rows_v, acc.at[pl.ds(rbase + k * CH, CH)])

    plsc.subcore_barrier()

    @pl.loop(0, NCH)
    def _chunk(j):
        off = base + j * CH
        pltpu.sync_copy(idx_hbm.at[pl.ds(off, CH)], idx_v)
        pltpu.sync_copy(vals_hbm.at[pl.ds(off, CH)], rows_v)
        pltpu.sync_copy(rows_v, acc.at[idx_v], add=True)

    plsc.subcore_barrier()

    @pl.loop(0, RT // CH)
    def _w(k):
        sl = pl.ds(rbase + k * CH, CH)
        pltpu.sync_copy(acc.at[sl], rows_v)

        @pl.when(cid == 0)
        def _():
            pltpu.sync_copy(rows_v, out0_hbm.at[sl])

        @pl.when(cid == 1)
        def _():
            pltpu.sync_copy(rows_v, out1_hbm.at[sl])


def _sc_scatter(vals, idx, z128):
    fn = pl.kernel(
        _sc_scatter_body,
        out_type=(jax.ShapeDtypeStruct((NPAD, L), _f32),
                  jax.ShapeDtypeStruct((NPAD, L), _f32)),
        mesh=_mesh(),
        scratch_types=[
            pltpu.VMEM((CH,), jnp.int32),
            pltpu.VMEM((CH, L), _f32),
            pltpu.VMEM_SHARED((NPAD, L), _f32),
        ],
    )
    return fn(vals, idx, z128)


def _sc_scatter_enc_body(vals_hbm, src_hbm, dst_hbm, z128_hbm, z16_hbm, ones_hbm,
                         s0_hbm, s1_hbm, c0_hbm, c1_hbm, d0_hbm, d1_hbm,
                         idxs, idxd, rows_v, ones_v, accS, accC, accD):
    cid = lax.axis_index("c")
    sid = lax.axis_index("s")
    wid = sid * NC + cid
    base = wid * EW
    rbase = sid * RT

    pltpu.sync_copy(z128_hbm, rows_v)
    pltpu.sync_copy(z16_hbm, ones_v)

    @pl.loop(0, RT // CH)
    def _z(k):
        sl = pl.ds(rbase + k * CH, CH)
        pltpu.sync_copy(rows_v, accS.at[sl])
        pltpu.sync_copy(ones_v, accC.at[sl])
        pltpu.sync_copy(ones_v, accD.at[sl])

    pltpu.sync_copy(ones_hbm, ones_v)
    plsc.subcore_barrier()

    @pl.loop(0, NCH)
    def _chunk(j):
        off = base + j * CH
        pltpu.sync_copy(src_hbm.at[pl.ds(off, CH)], idxs)
        pltpu.sync_copy(dst_hbm.at[pl.ds(off, CH)], idxd)
        pltpu.sync_copy(vals_hbm.at[pl.ds(off, CH)], rows_v)
        pltpu.sync_copy(rows_v, accS.at[idxs], add=True)
        pltpu.sync_copy(ones_v, accC.at[idxs], add=True)
        pltpu.sync_copy(ones_v, accD.at[idxd], add=True)

    plsc.subcore_barrier()

    @pl.loop(0, RT // CH)
    def _w(k):
        sl = pl.ds(rbase + k * CH, CH)
        pltpu.sync_copy(accS.at[sl], rows_v)
        pltpu.sync_copy(accC.at[sl], ones_v)

        @pl.when(cid == 0)
        def _():
            pltpu.sync_copy(rows_v, s0_hbm.at[sl])
            pltpu.sync_copy(ones_v, c0_hbm.at[sl])

        @pl.when(cid == 1)
        def _():
            pltpu.sync_copy(rows_v, s1_hbm.at[sl])
            pltpu.sync_copy(ones_v, c1_hbm.at[sl])

        pltpu.sync_copy(accD.at[sl], ones_v)

        @pl.when(cid == 0)
        def _():
            pltpu.sync_copy(ones_v, d0_hbm.at[sl])

        @pl.when(cid == 1)
        def _():
            pltpu.sync_copy(ones_v, d1_hbm.at[sl])


def _sc_scatter_enc(vals, src, dst, z128, z16, ones16):
    fn = pl.kernel(
        _sc_scatter_enc_body,
        out_type=(
            jax.ShapeDtypeStruct((NPAD, L), _f32),
            jax.ShapeDtypeStruct((NPAD, L), _f32),
            jax.ShapeDtypeStruct((NPAD, 16), _f32),
            jax.ShapeDtypeStruct((NPAD, 16), _f32),
            jax.ShapeDtypeStruct((NPAD, 16), _f32),
            jax.ShapeDtypeStruct((NPAD, 16), _f32),
        ),
        mesh=_mesh(),
        scratch_types=[
            pltpu.VMEM((CH,), jnp.int32),
            pltpu.VMEM((CH,), jnp.int32),
            pltpu.VMEM((CH, L), _f32),
            pltpu.VMEM((CH, 16), _f32),
            pltpu.VMEM_SHARED((NPAD, L), _f32),
            pltpu.VMEM_SHARED((NPAD, 16), _f32),
            pltpu.VMEM_SHARED((NPAD, 16), _f32),
        ],
    )
    return fn(vals, src, dst, z128, z16, ones16)


def _ln(t, g, b):
    m = jnp.mean(t, axis=-1, keepdims=True)
    v = jnp.mean((t - m) * (t - m), axis=-1, keepdims=True)
    return (t - m) * lax.rsqrt(v + 1e-5) * g + b


def _dot(a, b):
    return jnp.dot(a, b, preferred_element_type=_f32)


def _edge_enc_body(ef, w1, b1, w2, b2, w3, b3, lg, lb, out):
    t = _dot(ef[...], w1[...]) + b1[...]
    t = _dot(t, w2[...]) + b2[...]
    t = _dot(t, w3[...]) + b3[...]
    out[...] = _ln(t, lg[...], lb[...])


def _edge_step_body(e_ref, g_ref, w1c, w2, b2, w3, b3, lg, lb, enew_ref, enext_ref):
    e = e_ref[...]
    h = g_ref[...] + _dot(e, w1c[...])
    h = _dot(h, w2[...]) + b2[...]
    t = _dot(h, w3[...]) + b3[...]
    en = _ln(t, lg[...], lb[...])
    enew_ref[...] = en
    enext_ref[...] = e + en


def _node_first_body(nf, w1, b1, w2, b2, w3, b3, lg, lb,
                     s0, s1, c0, c1, w1a, w1b, be1,
                     x_ref, xa_ref, xb_ref):
    t = _dot(nf[...], w1[...]) + b1[...]
    t = _dot(t, w2[...]) + b2[...]
    t = _dot(t, w3[...]) + b3[...]
    xe = _ln(t, lg[...], lb[...])
    cnt = jnp.maximum(c0[:, 0:1] + c1[:, 0:1], 1.0)
    x = xe + (s0[...] + s1[...]) / cnt
    x_ref[...] = x
    xa_ref[...] = _dot(x, w1a[...]) + be1[...]
    xb_ref[...] = _dot(x, w1b[...])


def _node_step_body(x_ref, q0, q1, d0, d1,
                    v1a, v1b, cb1, v2, cb2, v3, cb3, lg, lb,
                    w1a, w1b, be1,
                    xn_ref, xa_ref, xb_ref):
    x = x_ref[...]
    cnt = jnp.maximum(d0[:, 0:1] + d1[:, 0:1], 1.0)
    agg = (q0[...] + q1[...]) / cnt
    u = _dot(x, v1a[...]) + _dot(agg, v1b[...]) + cb1[...]
    u = _dot(u, v2[...]) + cb2[...]
    u = _dot(u, v3[...]) + cb3[...]
    xn = x + _ln(u, lg[...], lb[...])
    xn_ref[...] = xn
    xa_ref[...] = _dot(xn, w1a[...]) + be1[...]
    xb_ref[...] = _dot(xn, w1b[...])


def _node_last_body(x_ref, q0, q1, d0, d1,
                    v1a, v1b, cb1, v2, cb2, v3, cb3, lg, lb,
                    dw1, db1, dw2, db2, dw3, db3,
                    out_ref):
    x = x_ref[...]
    cnt = jnp.maximum(d0[:, 0:1] + d1[:, 0:1], 1.0)
    agg = (q0[...] + q1[...]) / cnt
    u = _dot(x, v1a[...]) + _dot(agg, v1b[...]) + cb1[...]
    u = _dot(u, v2[...]) + cb2[...]
    u = _dot(u, v3[...]) + cb3[...]
    xn = x + _ln(u, lg[...], lb[...])
    t = _dot(xn, dw1[...]) + db1[...]
    t = _dot(t, dw2[...]) + db2[...]
    out_ref[...] = _dot(t, dw3[...]) + db3[...]


def _full(shape):
    return pl.BlockSpec(shape, lambda i: (0,) * len(shape))


def _rows(nrow, ncol):
    return pl.BlockSpec((nrow, ncol), lambda i: (i, 0))


def _call_edge_enc(ef, w):
    grid = (E // BE,)
    return pl.pallas_call(
        _edge_enc_body,
        grid=grid,
        in_specs=[_rows(BE, 16)] + [_full(a.shape) for a in w],
        out_specs=_rows(BE, L),
        out_shape=jax.ShapeDtypeStruct((E, L), _f32),
    )(ef, *w)


def _call_edge_step(e, g, w):
    grid = (E // BE,)
    return pl.pallas_call(
        _edge_step_body,
        grid=grid,
        in_specs=[_rows(BE, L), _rows(BE, L)] + [_full(a.shape) for a in w],
        out_specs=(_rows(BE, L), _rows(BE, L)),
        out_shape=(jax.ShapeDtypeStruct((E, L), _f32),
                   jax.ShapeDtypeStruct((E, L), _f32)),
    )(e, g, *w)


def _call_node_first(nf, encw, s, c, projw):
    grid = (N // BN,)
    specs = ([_rows(BN, L)] + [_full(a.shape) for a in encw]
             + [_rows(BN, L), _rows(BN, L), _rows(BN, 16), _rows(BN, 16)]
             + [_full(a.shape) for a in projw])
    return pl.pallas_call(
        _node_first_body,
        grid=grid,
        in_specs=specs,
        out_specs=(_rows(BN, L), _rows(BN, L), _rows(BN, L)),
        out_shape=(jax.ShapeDtypeStruct((N, L), _f32),) * 3,
    )(nf, *encw, s[0], s[1], c[0], c[1], *projw)


def _call_node_step(x, q, d, nodew, projw):
    grid = (N // BN,)
    specs = ([_rows(BN, L), _rows(BN, L), _rows(BN, L), _rows(BN, 16), _rows(BN, 16)]
             + [_full(a.shape) for a in nodew]
             + [_full(a.shape) for a in projw])
    return pl.pallas_call(
        _node_step_body,
        grid=grid,
        in_specs=specs,
        out_specs=(_rows(BN, L), _rows(BN, L), _rows(BN, L)),
        out_shape=(jax.ShapeDtypeStruct((N, L), _f32),) * 3,
    )(x, q[0], q[1], d[0], d[1], *nodew, *projw)


def _call_node_last(x, q, d, nodew, decw):
    grid = (N // BN,)
    specs = ([_rows(BN, L), _rows(BN, L), _rows(BN, L), _rows(BN, 16), _rows(BN, 16)]
             + [_full(a.shape) for a in nodew]
             + [_full(a.shape) for a in decw])
    return pl.pallas_call(
        _node_last_body,
        grid=grid,
        in_specs=specs,
        out_specs=_rows(BN, 3),
        out_shape=jax.ShapeDtypeStruct((N, 3), _f32),
    )(x, q[0], q[1], d[0], d[1], *nodew, *decw)


def _row(v):
    return v.reshape(1, -1)


def _block_weights(blk):
    (w1, b1), (w2, b2), (w3, b3) = blk["mlp"]
    lg, lb = blk["ln"]
    return [w1, _row(b1), w2, _row(b2), w3, _row(b3), _row(lg), _row(lb)]


def kernel(node_feat, edge_feat, edge_index, params):
    src = edge_index[0].astype(jnp.int32)
    dst = edge_index[1].astype(jnp.int32)

    z128 = jnp.zeros((CH, L), _f32)
    z16 = jnp.zeros((CH, 16), _f32)
    ones16 = jnp.ones((CH, 16), _f32)

    encw_e = _block_weights(params["edge_enc"])
    encw_n = _block_weights(params["node_enc"])

    edge_w, node_w, proj_w = [], [], []
    for p in params["proc"]:
        (w1, b1), (w2, b2), (w3, b3) = p["edge"]["mlp"]
        lg, lb = p["edge"]["ln"]
        proj_w.append([w1[:L], w1[L:2 * L], _row(b1)])
        edge_w.append([w1[2 * L:], w2, _row(b2), w3, _row(b3), _row(lg), _row(lb)])
        (v1, c1), (v2, c2), (v3, c3) = p["node"]["mlp"]
        ng, nb = p["node"]["ln"]
        node_w.append([v1[:L], v1[L:], _row(c1), v2, _row(c2), v3, _row(c3),
                       _row(ng), _row(nb)])
    (dw1, db1), (dw2, db2), (dw3, db3) = params["dec"]
    decw = [dw1, _row(db1), dw2, _row(db2), dw3, _row(db3)]

    def _jnp_seg(vals, idx):
        sfull = jax.ops.segment_sum(vals, idx, num_segments=N)
        zpad = jnp.zeros((NPAD - N, vals.shape[1]), _f32)
        return jnp.concatenate([sfull, zpad], 0), jnp.zeros((NPAD, vals.shape[1]), _f32)

    e = _call_edge_enc(edge_feat, encw_e)
    s0, s1 = _jnp_seg(e, src)
    ones_e = jnp.ones((E, 16), _f32)
    c0, c1 = _jnp_seg(ones_e, src)
    d0, d1 = _jnp_seg(ones_e, dst)
    s, c, d = (s0, s1), (c0, c1), (d0, d1)
    x, xa, xb = _call_node_first(node_feat, encw_n, s, c, proj_w[0])

    for i in range(4):
        g = _sc_gather(xa, xb, src, dst)
        e_new, e = _call_edge_step(e, g, edge_w[i])
        q0, q1 = _sc_scatter(e_new, dst, z128)
        q = (q0, q1)
        if i < 3:
            x, xa, xb = _call_node_step(x, q, d, node_w[i], proj_w[i + 1])
        else:
            out = _call_node_last(x, q, d, node_w[i], decw)
    return out


# speedup vs baseline: 2.6100x; 1.2870x over previous
import functools

import jax
import jax.numpy as jnp
from jax import lax
from jax.experimental import pallas as pl
from jax.experimental.pallas import tpu as pltpu
from jax.experimental.pallas import tpu_sc as plsc

N = 10000
E = 320000
L = 128
NC = 2
NS = 16
NW = NC * NS
EW = E // NW
CH = 80
NCH = EW // CH
RT = 640
NPAD = NS * RT

BE = 2000
BN = 2000

_f32 = jnp.float32


def _mesh():
    return plsc.VectorSubcoreMesh(
        core_axis_name="c", subcore_axis_name="s", num_cores=NC, num_subcores=NS
    )


def _sc_gather_body(xa_hbm, xb_hbm, src_hbm, dst_hbm, g_hbm,
                    ia, ib, ra, rb, sema, semb):
    cid = lax.axis_index("c")
    sid = lax.axis_index("s")
    wid = sid * NC + cid
    base = wid * EW

    @pl.loop(0, NCH)
    def _chunk(j):
        off = base + j * CH
        pltpu.sync_copy(src_hbm.at[pl.ds(off, CH)], ia)
        pltpu.sync_copy(dst_hbm.at[pl.ds(off, CH)], ib)
        da = pltpu.async_copy(xa_hbm.at[ia], ra, sema)
        db = pltpu.async_copy(xb_hbm.at[ib], rb, semb)
        da.wait()
        db.wait()

        @pl.loop(0, CH)
        def _row(r):
            for k in range(L // 16):
                s = pl.ds(k * 16, 16)
                ra[r, s] = ra[r, s] + rb[r, s]

        pltpu.sync_copy(ra, g_hbm.at[pl.ds(off, CH)])


def _sc_gather(xa, xb, src, dst):
    fn = pl.kernel(
        _sc_gather_body,
        out_type=jax.ShapeDtypeStruct((E, L), _f32),
        mesh=_mesh(),
        scratch_types=[
            pltpu.VMEM((CH,), jnp.int32),
            pltpu.VMEM((CH,), jnp.int32),
            pltpu.VMEM((CH, L), _f32),
            pltpu.VMEM((CH, L), _f32),
            pltpu.SemaphoreType.DMA,
            pltpu.SemaphoreType.DMA,
        ],
    )
    return fn(xa, xb, src, dst)


def _sc_scatter_body(vals_hbm, idx_hbm, z128_hbm, out0_hbm, out1_hbm,
                     idx_v, rows_v, acc):
    cid = lax.axis_index("c")
    sid = lax.axis_index("s")
    wid = sid * NC + cid
    base = wid * EW
    rbase = sid * RT

    pltpu.sync_copy(z128_hbm, rows_v)

    @pl.loop(0, RT // CH)
    def _z(k):
        pltpu.sync_copy(rows_v, acc.at[pl.ds(rbase + k * CH, CH)])

    plsc.subcore_barrier()

    @pl.loop(0, NCH)
    def _chunk(j):
        off = base + j * CH
        pltpu.sync_copy(idx_hbm.at[pl.ds(off, CH)], idx_v)
        pltpu.sync_copy(vals_hbm.at[pl.ds(off, CH)], rows_v)
        pltpu.sync_copy(rows_v, acc.at[idx_v], add=True)

    plsc.subcore_barrier()

    @pl.loop(0, RT // CH)
    def _w(k):
        sl = pl.ds(rbase + k * CH, CH)
        pltpu.sync_copy(acc.at[sl], rows_v)

        @pl.when(cid == 0)
        def _():
            pltpu.sync_copy(rows_v, out0_hbm.at[sl])

        @pl.when(cid == 1)
        def _():
            pltpu.sync_copy(rows_v, out1_hbm.at[sl])


def _sc_scatter(vals, idx, z128):
    fn = pl.kernel(
        _sc_scatter_body,
        out_type=(jax.ShapeDtypeStruct((NPAD, L), _f32),
                  jax.ShapeDtypeStruct((NPAD, L), _f32)),
        mesh=_mesh(),
        scratch_types=[
            pltpu.VMEM((CH,), jnp.int32),
            pltpu.VMEM((CH, L), _f32),
            pltpu.VMEM_SHARED((NPAD, L), _f32),
        ],
    )
    return fn(vals, idx, z128)


def _sc_count_body(idx_hbm, ones_hbm, z128_hbm, out0_hbm, out1_hbm,
                   idx_v, rows_v, acc):
    cid = lax.axis_index("c")
    sid = lax.axis_index("s")
    wid = sid * NC + cid
    base = wid * EW
    rbase = sid * RT

    pltpu.sync_copy(z128_hbm, rows_v)

    @pl.loop(0, RT // CH)
    def _z(k):
        pltpu.sync_copy(rows_v, acc.at[pl.ds(rbase + k * CH, CH)])

    pltpu.sync_copy(ones_hbm, rows_v)
    plsc.subcore_barrier()

    @pl.loop(0, NCH)
    def _chunk(j):
        off = base + j * CH
        pltpu.sync_copy(idx_hbm.at[pl.ds(off, CH)], idx_v)
        pltpu.sync_copy(rows_v, acc.at[idx_v], add=True)

    plsc.subcore_barrier()

    @pl.loop(0, RT // CH)
    def _w(k):
        sl = pl.ds(rbase + k * CH, CH)
        pltpu.sync_copy(acc.at[sl], rows_v)

        @pl.when(cid == 0)
        def _():
            pltpu.sync_copy(rows_v, out0_hbm.at[sl])

        @pl.when(cid == 1)
        def _():
            pltpu.sync_copy(rows_v, out1_hbm.at[sl])

        pltpu.sync_copy(ones_hbm, rows_v)


def _sc_count(idx, ones128, z128):
    fn = pl.kernel(
        _sc_count_body,
        out_type=(jax.ShapeDtypeStruct((NPAD, L), _f32),
                  jax.ShapeDtypeStruct((NPAD, L), _f32)),
        mesh=_mesh(),
        scratch_types=[
            pltpu.VMEM((CH,), jnp.int32),
            pltpu.VMEM((CH, L), _f32),
            pltpu.VMEM_SHARED((NPAD, L), _f32),
        ],
    )
    return fn(idx, ones128, z128)


def _ln(t, g, b):
    m = jnp.mean(t, axis=-1, keepdims=True)
    v = jnp.mean((t - m) * (t - m), axis=-1, keepdims=True)
    return (t - m) * lax.rsqrt(v + 1e-5) * g + b


def _dot(a, b):
    return jnp.dot(a, b, preferred_element_type=_f32)


def _edge_enc_body(ef, w1, b1, w2, b2, w3, b3, lg, lb, out):
    t = _dot(ef[...], w1[...]) + b1[...]
    t = _dot(t, w2[...]) + b2[...]
    t = _dot(t, w3[...]) + b3[...]
    out[...] = _ln(t, lg[...], lb[...])


def _edge_step_body(e_ref, g_ref, w1c, w2, b2, w3, b3, lg, lb, enew_ref, enext_ref):
    e = e_ref[...]
    h = g_ref[...] + _dot(e, w1c[...])
    h = _dot(h, w2[...]) + b2[...]
    t = _dot(h, w3[...]) + b3[...]
    en = _ln(t, lg[...], lb[...])
    enew_ref[...] = en
    enext_ref[...] = e + en


def _node_first_body(nf, w1, b1, w2, b2, w3, b3, lg, lb,
                     s0, s1, c0, c1, w1a, w1b, be1,
                     x_ref, xa_ref, xb_ref):
    t = _dot(nf[...], w1[...]) + b1[...]
    t = _dot(t, w2[...]) + b2[...]
    t = _dot(t, w3[...]) + b3[...]
    xe = _ln(t, lg[...], lb[...])
    cnt = jnp.maximum(c0[:, 0:1] + c1[:, 0:1], 1.0)
    x = xe + (s0[...] + s1[...]) / cnt
    x_ref[...] = x
    xa_ref[...] = _dot(x, w1a[...]) + be1[...]
    xb_ref[...] = _dot(x, w1b[...])


def _node_step_body(x_ref, q0, q1, d0, d1,
                    v1a, v1b, cb1, v2, cb2, v3, cb3, lg, lb,
                    w1a, w1b, be1,
                    xn_ref, xa_ref, xb_ref):
    x = x_ref[...]
    cnt = jnp.maximum(d0[:, 0:1] + d1[:, 0:1], 1.0)
    agg = (q0[...] + q1[...]) / cnt
    u = _dot(x, v1a[...]) + _dot(agg, v1b[...]) + cb1[...]
    u = _dot(u, v2[...]) + cb2[...]
    u = _dot(u, v3[...]) + cb3[...]
    xn = x + _ln(u, lg[...], lb[...])
    xn_ref[...] = xn
    xa_ref[...] = _dot(xn, w1a[...]) + be1[...]
    xb_ref[...] = _dot(xn, w1b[...])


def _node_last_body(x_ref, q0, q1, d0, d1,
                    v1a, v1b, cb1, v2, cb2, v3, cb3, lg, lb,
                    dw1, db1, dw2, db2, dw3, db3,
                    out_ref):
    x = x_ref[...]
    cnt = jnp.maximum(d0[:, 0:1] + d1[:, 0:1], 1.0)
    agg = (q0[...] + q1[...]) / cnt
    u = _dot(x, v1a[...]) + _dot(agg, v1b[...]) + cb1[...]
    u = _dot(u, v2[...]) + cb2[...]
    u = _dot(u, v3[...]) + cb3[...]
    xn = x + _ln(u, lg[...], lb[...])
    t = _dot(xn, dw1[...]) + db1[...]
    t = _dot(t, dw2[...]) + db2[...]
    out_ref[...] = _dot(t, dw3[...]) + db3[...]


def _full(shape):
    return pl.BlockSpec(shape, lambda i: (0,) * len(shape))


def _rows(nrow, ncol):
    return pl.BlockSpec((nrow, ncol), lambda i: (i, 0))


def _call_edge_enc(ef, w):
    grid = (E // BE,)
    return pl.pallas_call(
        _edge_enc_body,
        grid=grid,
        in_specs=[_rows(BE, 16)] + [_full(a.shape) for a in w],
        out_specs=_rows(BE, L),
        out_shape=jax.ShapeDtypeStruct((E, L), _f32),
    )(ef, *w)


def _call_edge_step(e, g, w):
    grid = (E // BE,)
    return pl.pallas_call(
        _edge_step_body,
        grid=grid,
        in_specs=[_rows(BE, L), _rows(BE, L)] + [_full(a.shape) for a in w],
        out_specs=(_rows(BE, L), _rows(BE, L)),
        out_shape=(jax.ShapeDtypeStruct((E, L), _f32),
                   jax.ShapeDtypeStruct((E, L), _f32)),
    )(e, g, *w)


def _call_node_first(nf, encw, s, c, projw):
    grid = (N // BN,)
    specs = ([_rows(BN, L)] + [_full(a.shape) for a in encw]
             + [_rows(BN, L), _rows(BN, L), _rows(BN, L), _rows(BN, L)]
             + [_full(a.shape) for a in projw])
    return pl.pallas_call(
        _node_first_body,
        grid=grid,
        in_specs=specs,
        out_specs=(_rows(BN, L), _rows(BN, L), _rows(BN, L)),
        out_shape=(jax.ShapeDtypeStruct((N, L), _f32),) * 3,
    )(nf, *encw, s[0], s[1], c[0], c[1], *projw)


def _call_node_step(x, q, d, nodew, projw):
    grid = (N // BN,)
    specs = ([_rows(BN, L), _rows(BN, L), _rows(BN, L), _rows(BN, L), _rows(BN, L)]
             + [_full(a.shape) for a in nodew]
             + [_full(a.shape) for a in projw])
    return pl.pallas_call(
        _node_step_body,
        grid=grid,
        in_specs=specs,
        out_specs=(_rows(BN, L), _rows(BN, L), _rows(BN, L)),
        out_shape=(jax.ShapeDtypeStruct((N, L), _f32),) * 3,
    )(x, q[0], q[1], d[0], d[1], *nodew, *projw)


def _call_node_last(x, q, d, nodew, decw):
    grid = (N // BN,)
    specs = ([_rows(BN, L), _rows(BN, L), _rows(BN, L), _rows(BN, L), _rows(BN, L)]
             + [_full(a.shape) for a in nodew]
             + [_full(a.shape) for a in decw])
    return pl.pallas_call(
        _node_last_body,
        grid=grid,
        in_specs=specs,
        out_specs=_rows(BN, 3),
        out_shape=jax.ShapeDtypeStruct((N, 3), _f32),
    )(x, q[0], q[1], d[0], d[1], *nodew, *decw)


def _row(v):
    return v.reshape(1, -1)


def _block_weights(blk):
    (w1, b1), (w2, b2), (w3, b3) = blk["mlp"]
    lg, lb = blk["ln"]
    return [w1, _row(b1), w2, _row(b2), w3, _row(b3), _row(lg), _row(lb)]


def kernel(node_feat, edge_feat, edge_index, params):
    src = edge_index[0].astype(jnp.int32)
    dst = edge_index[1].astype(jnp.int32)

    z128 = jnp.zeros((CH, L), _f32)
    ones128 = jnp.ones((CH, L), _f32)

    encw_e = _block_weights(params["edge_enc"])
    encw_n = _block_weights(params["node_enc"])

    edge_w, node_w, proj_w = [], [], []
    for p in params["proc"]:
        (w1, b1), (w2, b2), (w3, b3) = p["edge"]["mlp"]
        lg, lb = p["edge"]["ln"]
        proj_w.append([w1[:L], w1[L:2 * L], _row(b1)])
        edge_w.append([w1[2 * L:], w2, _row(b2), w3, _row(b3), _row(lg), _row(lb)])
        (v1, c1), (v2, c2), (v3, c3) = p["node"]["mlp"]
        ng, nb = p["node"]["ln"]
        node_w.append([v1[:L], v1[L:], _row(c1), v2, _row(c2), v3, _row(c3),
                       _row(ng), _row(nb)])
    (dw1, db1), (dw2, db2), (dw3, db3) = params["dec"]
    decw = [dw1, _row(db1), dw2, _row(db2), dw3, _row(db3)]

    e = _call_edge_enc(edge_feat, encw_e)
    s0, s1 = _sc_scatter(e, src, z128)
    c0, c1 = _sc_count(src, ones128, z128)
    d0, d1 = _sc_count(dst, ones128, z128)
    s, c, d = (s0, s1), (c0, c1), (d0, d1)
    x, xa, xb = _call_node_first(node_feat, encw_n, s, c, proj_w[0])

    for i in range(4):
        g = _sc_gather(xa, xb, src, dst)
        e_new, e = _call_edge_step(e, g, edge_w[i])
        q0, q1 = _sc_scatter(e_new, dst, z128)
        q = (q0, q1)
        if i < 3:
            x, xa, xb = _call_node_step(x, q, d, node_w[i], proj_w[i + 1])
        else:
            out = _call_node_last(x, q, d, node_w[i], decw)
    return out


# speedup vs baseline: 3.7641x; 1.4422x over previous
import functools

import jax
import jax.numpy as jnp
from jax import lax
from jax.experimental import pallas as pl
from jax.experimental.pallas import tpu as pltpu
from jax.experimental.pallas import tpu_sc as plsc

N = 10000
E = 320000
L = 128
NC = 2
NS = 16
NW = NC * NS
EW = E // NW
CH = 80
NCH = EW // CH
RT = 640
NPAD = NS * RT

BE = 2000
BN = 2000

_f32 = jnp.float32


def _mesh():
    return plsc.VectorSubcoreMesh(
        core_axis_name="c", subcore_axis_name="s", num_cores=NC, num_subcores=NS
    )


def _sc_gather_body(xa_hbm, xb_hbm, src_hbm, dst_hbm, ga_hbm, gb_hbm,
                    ia, ib, ra0, rb0, ra1, rb1,
                    sa0, sb0, sa1, sb1, pa0, pb0, pa1, pb1):
    cid = lax.axis_index("c")
    sid = lax.axis_index("s")
    wid = sid * NC + cid
    base = wid * EW

    pltpu.sync_copy(src_hbm.at[pl.ds(base, EW)], ia)
    pltpu.sync_copy(dst_hbm.at[pl.ds(base, EW)], ib)

    ras = (ra0, ra1)
    rbs = (rb0, rb1)
    sga = (sa0, sa1)
    sgb = (sb0, sb1)
    ssa = (pa0, pa1)
    ssb = (pb0, pb1)

    def idxa(c):
        return ia.at[pl.ds(c * CH, CH)]

    def idxb(c):
        return ib.at[pl.ds(c * CH, CH)]

    pltpu.async_copy(xa_hbm.at[idxa(0)], ra0, sa0)
    pltpu.async_copy(xb_hbm.at[idxb(0)], rb0, sb0)

    @pl.loop(0, NCH - 1, step=2)
    def _pair(j):
        for b in range(2):
            c = j + b
            nb = 1 - b

            @pl.when(c > 0)
            def _():
                pltpu.make_async_copy(
                    ras[nb], ga_hbm.at[pl.ds(base, CH)], ssa[nb]).wait()
                pltpu.make_async_copy(
                    rbs[nb], gb_hbm.at[pl.ds(base, CH)], ssb[nb]).wait()

            pltpu.async_copy(xa_hbm.at[idxa(c + 1)], ras[nb], sga[nb])
            pltpu.async_copy(xb_hbm.at[idxb(c + 1)], rbs[nb], sgb[nb])

            pltpu.make_async_copy(xa_hbm.at[idxa(c)], ras[b], sga[b]).wait()
            pltpu.make_async_copy(xb_hbm.at[idxb(c)], rbs[b], sgb[b]).wait()
            off = base + c * CH
            pltpu.async_copy(ras[b], ga_hbm.at[pl.ds(off, CH)], ssa[b])
            pltpu.async_copy(rbs[b], gb_hbm.at[pl.ds(off, CH)], ssb[b])

    cl = NCH - 1
    pltpu.make_async_copy(ra1, ga_hbm.at[pl.ds(base, CH)], pa1).wait()
    pltpu.make_async_copy(rb1, gb_hbm.at[pl.ds(base, CH)], pb1).wait()
    pltpu.make_async_copy(xa_hbm.at[idxa(cl)], ra0, sa0).wait()
    pltpu.make_async_copy(xb_hbm.at[idxb(cl)], rb0, sb0).wait()
    off = base + cl * CH
    pltpu.sync_copy(ra0, ga_hbm.at[pl.ds(off, CH)])
    pltpu.sync_copy(rb0, gb_hbm.at[pl.ds(off, CH)])


def _sc_gather(xa, xb, src, dst):
    fn = pl.kernel(
        _sc_gather_body,
        out_type=(jax.ShapeDtypeStruct((E, L), _f32),
                  jax.ShapeDtypeStruct((E, L), _f32)),
        mesh=_mesh(),
        scratch_types=[
            pltpu.VMEM((EW,), jnp.int32),
            pltpu.VMEM((EW,), jnp.int32),
            pltpu.VMEM((CH, L), _f32),
            pltpu.VMEM((CH, L), _f32),
            pltpu.VMEM((CH, L), _f32),
            pltpu.VMEM((CH, L), _f32),
        ] + [pltpu.SemaphoreType.DMA] * 8,
    )
    return fn(xa, xb, src, dst)


def _sc_scatter_body(vals_hbm, idx_hbm, z128_hbm, out0_hbm, out1_hbm,
                     idx_full, v0, v1, sv0, sv1, acc):
    cid = lax.axis_index("c")
    sid = lax.axis_index("s")
    wid = sid * NC + cid
    base = wid * EW
    rbase = sid * RT

    pltpu.sync_copy(idx_hbm.at[pl.ds(base, EW)], idx_full)

    pltpu.sync_copy(z128_hbm, v0)

    @pl.loop(0, RT // CH)
    def _z(k):
        pltpu.sync_copy(v0, acc.at[pl.ds(rbase + k * CH, CH)])

    pltpu.async_copy(vals_hbm.at[pl.ds(base, CH)], v0, sv0)
    plsc.subcore_barrier()

    vs = (v0, v1)
    svs = (sv0, sv1)

    @pl.loop(0, NCH - 1, step=2)
    def _pair(j):
        for b in range(2):
            c = j + b
            nb = 1 - b
            pltpu.async_copy(
                vals_hbm.at[pl.ds(base + (c + 1) * CH, CH)], vs[nb], svs[nb])
            pltpu.make_async_copy(
                vals_hbm.at[pl.ds(base, CH)], vs[b], svs[b]).wait()
            pltpu.sync_copy(
                vs[b], acc.at[idx_full.at[pl.ds(c * CH, CH)]], add=True)

    cl = NCH - 1
    pltpu.make_async_copy(vals_hbm.at[pl.ds(base, CH)], v0, sv0).wait()
    pltpu.sync_copy(v0, acc.at[idx_full.at[pl.ds(cl * CH, CH)]], add=True)

    plsc.subcore_barrier()

    @pl.loop(0, RT // CH)
    def _w(k):
        sl = pl.ds(rbase + k * CH, CH)
        pltpu.sync_copy(acc.at[sl], v0)

        @pl.when(cid == 0)
        def _():
            pltpu.sync_copy(v0, out0_hbm.at[sl])

        @pl.when(cid == 1)
        def _():
            pltpu.sync_copy(v0, out1_hbm.at[sl])


def _sc_scatter(vals, idx, z128):
    fn = pl.kernel(
        _sc_scatter_body,
        out_type=(jax.ShapeDtypeStruct((NPAD, L), _f32),
                  jax.ShapeDtypeStruct((NPAD, L), _f32)),
        mesh=_mesh(),
        scratch_types=[
            pltpu.VMEM((EW,), jnp.int32),
            pltpu.VMEM((CH, L), _f32),
            pltpu.VMEM((CH, L), _f32),
            pltpu.SemaphoreType.DMA,
            pltpu.SemaphoreType.DMA,
            pltpu.VMEM_SHARED((NPAD, L), _f32),
        ],
    )
    return fn(vals, idx, z128)


def _sc_count_body(idx_hbm, ones_hbm, z128_hbm, out0_hbm, out1_hbm,
                   idx_full, rows_v, acc):
    cid = lax.axis_index("c")
    sid = lax.axis_index("s")
    wid = sid * NC + cid
    base = wid * EW
    rbase = sid * RT

    pltpu.sync_copy(idx_hbm.at[pl.ds(base, EW)], idx_full)
    pltpu.sync_copy(z128_hbm, rows_v)

    @pl.loop(0, RT // CH)
    def _z(k):
        pltpu.sync_copy(rows_v, acc.at[pl.ds(rbase + k * CH, CH)])

    pltpu.sync_copy(ones_hbm, rows_v)
    plsc.subcore_barrier()

    @pl.loop(0, NCH)
    def _chunk(j):
        pltpu.sync_copy(
            rows_v, acc.at[idx_full.at[pl.ds(j * CH, CH)]], add=True)

    plsc.subcore_barrier()

    @pl.loop(0, RT // CH)
    def _w(k):
        sl = pl.ds(rbase + k * CH, CH)
        pltpu.sync_copy(acc.at[sl], rows_v)

        @pl.when(cid == 0)
        def _():
            pltpu.sync_copy(rows_v, out0_hbm.at[sl])

        @pl.when(cid == 1)
        def _():
            pltpu.sync_copy(rows_v, out1_hbm.at[sl])

        pltpu.sync_copy(ones_hbm, rows_v)


def _sc_count(idx, ones128, z128):
    fn = pl.kernel(
        _sc_count_body,
        out_type=(jax.ShapeDtypeStruct((NPAD, L), _f32),
                  jax.ShapeDtypeStruct((NPAD, L), _f32)),
        mesh=_mesh(),
        scratch_types=[
            pltpu.VMEM((EW,), jnp.int32),
            pltpu.VMEM((CH, L), _f32),
            pltpu.VMEM_SHARED((NPAD, L), _f32),
        ],
    )
    return fn(idx, ones128, z128)


def _ln(t, g, b):
    m = jnp.mean(t, axis=-1, keepdims=True)
    v = jnp.mean((t - m) * (t - m), axis=-1, keepdims=True)
    return (t - m) * lax.rsqrt(v + 1e-5) * g + b


def _dot(a, b):
    return jnp.dot(a, b, preferred_element_type=_f32)


def _edge_enc_body(ef, w1, b1, w2, b2, w3, b3, lg, lb, out):
    t = _dot(ef[...], w1[...]) + b1[...]
    t = _dot(t, w2[...]) + b2[...]
    t = _dot(t, w3[...]) + b3[...]
    out[...] = _ln(t, lg[...], lb[...])


def _edge_step_body(e_ref, ga_ref, gb_ref, w1c, w2, b2, w3, b3, lg, lb,
                    enew_ref, enext_ref):
    e = e_ref[...]
    h = ga_ref[...] + gb_ref[...] + _dot(e, w1c[...])
    h = _dot(h, w2[...]) + b2[...]
    t = _dot(h, w3[...]) + b3[...]
    en = _ln(t, lg[...], lb[...])
    enew_ref[...] = en
    enext_ref[...] = e + en


def _node_first_body(nf, w1, b1, w2, b2, w3, b3, lg, lb,
                     s0, s1, c0, c1, w1a, w1b, be1,
                     x_ref, xa_ref, xb_ref):
    t = _dot(nf[...], w1[...]) + b1[...]
    t = _dot(t, w2[...]) + b2[...]
    t = _dot(t, w3[...]) + b3[...]
    xe = _ln(t, lg[...], lb[...])
    cnt = jnp.maximum(c0[:, 0:1] + c1[:, 0:1], 1.0)
    x = xe + (s0[...] + s1[...]) / cnt
    x_ref[...] = x
    xa_ref[...] = _dot(x, w1a[...]) + be1[...]
    xb_ref[...] = _dot(x, w1b[...])


def _node_step_body(x_ref, q0, q1, d0, d1,
                    v1a, v1b, cb1, v2, cb2, v3, cb3, lg, lb,
                    w1a, w1b, be1,
                    xn_ref, xa_ref, xb_ref):
    x = x_ref[...]
    cnt = jnp.maximum(d0[:, 0:1] + d1[:, 0:1], 1.0)
    agg = (q0[...] + q1[...]) / cnt
    u = _dot(x, v1a[...]) + _dot(agg, v1b[...]) + cb1[...]
    u = _dot(u, v2[...]) + cb2[...]
    u = _dot(u, v3[...]) + cb3[...]
    xn = x + _ln(u, lg[...], lb[...])
    xn_ref[...] = xn
    xa_ref[...] = _dot(xn, w1a[...]) + be1[...]
    xb_ref[...] = _dot(xn, w1b[...])


def _node_last_body(x_ref, q0, q1, d0, d1,
                    v1a, v1b, cb1, v2, cb2, v3, cb3, lg, lb,
                    dw1, db1, dw2, db2, dw3, db3,
                    out_ref):
    x = x_ref[...]
    cnt = jnp.maximum(d0[:, 0:1] + d1[:, 0:1], 1.0)
    agg = (q0[...] + q1[...]) / cnt
    u = _dot(x, v1a[...]) + _dot(agg, v1b[...]) + cb1[...]
    u = _dot(u, v2[...]) + cb2[...]
    u = _dot(u, v3[...]) + cb3[...]
    xn = x + _ln(u, lg[...], lb[...])
    t = _dot(xn, dw1[...]) + db1[...]
    t = _dot(t, dw2[...]) + db2[...]
    out_ref[...] = _dot(t, dw3[...]) + db3[...]


def _full(shape):
    return pl.BlockSpec(shape, lambda i: (0,) * len(shape))


def _rows(nrow, ncol):
    return pl.BlockSpec((nrow, ncol), lambda i: (i, 0))


def _call_edge_enc(ef, w):
    grid = (E // BE,)
    return pl.pallas_call(
        _edge_enc_body,
        grid=grid,
        in_specs=[_rows(BE, 16)] + [_full(a.shape) for a in w],
        out_specs=_rows(BE, L),
        out_shape=jax.ShapeDtypeStruct((E, L), _f32),
    )(ef, *w)


def _call_edge_step(e, ga, gb, w):
    grid = (E // BE,)
    return pl.pallas_call(
        _edge_step_body,
        grid=grid,
        in_specs=[_rows(BE, L), _rows(BE, L), _rows(BE, L)]
        + [_full(a.shape) for a in w],
        out_specs=(_rows(BE, L), _rows(BE, L)),
        out_shape=(jax.ShapeDtypeStruct((E, L), _f32),
                   jax.ShapeDtypeStruct((E, L), _f32)),
    )(e, ga, gb, *w)


def _call_node_first(nf, encw, s, c, projw):
    grid = (N // BN,)
    specs = ([_rows(BN, L)] + [_full(a.shape) for a in encw]
             + [_rows(BN, L), _rows(BN, L), _rows(BN, L), _rows(BN, L)]
             + [_full(a.shape) for a in projw])
    return pl.pallas_call(
        _node_first_body,
        grid=grid,
        in_specs=specs,
        out_specs=(_rows(BN, L), _rows(BN, L), _rows(BN, L)),
        out_shape=(jax.ShapeDtypeStruct((N, L), _f32),) * 3,
    )(nf, *encw, s[0], s[1], c[0], c[1], *projw)


def _call_node_step(x, q, d, nodew, projw):
    grid = (N // BN,)
    specs = ([_rows(BN, L), _rows(BN, L), _rows(BN, L), _rows(BN, L), _rows(BN, L)]
             + [_full(a.shape) for a in nodew]
             + [_full(a.shape) for a in projw])
    return pl.pallas_call(
        _node_step_body,
        grid=grid,
        in_specs=specs,
        out_specs=(_rows(BN, L), _rows(BN, L), _rows(BN, L)),
        out_shape=(jax.ShapeDtypeStruct((N, L), _f32),) * 3,
    )(x, q[0], q[1], d[0], d[1], *nodew, *projw)


def _call_node_last(x, q, d, nodew, decw):
    grid = (N // BN,)
    specs = ([_rows(BN, L), _rows(BN, L), _rows(BN, L), _rows(BN, L), _rows(BN, L)]
             + [_full(a.shape) for a in nodew]
             + [_full(a.shape) for a in decw])
    return pl.pallas_call(
        _node_last_body,
        grid=grid,
        in_specs=specs,
        out_specs=_rows(BN, 3),
        out_shape=jax.ShapeDtypeStruct((N, 3), _f32),
    )(x, q[0], q[1], d[0], d[1], *nodew, *decw)


def _row(v):
    return v.reshape(1, -1)


def _block_weights(blk):
    (w1, b1), (w2, b2), (w3, b3) = blk["mlp"]
    lg, lb = blk["ln"]
    return [w1, _row(b1), w2, _row(b2), w3, _row(b3), _row(lg), _row(lb)]


def kernel(node_feat, edge_feat, edge_index, params):
    src = edge_index[0].astype(jnp.int32)
    dst = edge_index[1].astype(jnp.int32)

    z128 = jnp.zeros((CH, L), _f32)
    ones128 = jnp.ones((CH, L), _f32)

    encw_e = _block_weights(params["edge_enc"])
    encw_n = _block_weights(params["node_enc"])

    edge_w, node_w, proj_w = [], [], []
    for p in params["proc"]:
        (w1, b1), (w2, b2), (w3, b3) = p["edge"]["mlp"]
        lg, lb = p["edge"]["ln"]
        proj_w.append([w1[:L], w1[L:2 * L], _row(b1)])
        edge_w.append([w1[2 * L:], w2, _row(b2), w3, _row(b3), _row(lg), _row(lb)])
        (v1, c1), (v2, c2), (v3, c3) = p["node"]["mlp"]
        ng, nb = p["node"]["ln"]
        node_w.append([v1[:L], v1[L:], _row(c1), v2, _row(c2), v3, _row(c3),
                       _row(ng), _row(nb)])
    (dw1, db1), (dw2, db2), (dw3, db3) = params["dec"]
    decw = [dw1, _row(db1), dw2, _row(db2), dw3, _row(db3)]

    e = _call_edge_enc(edge_feat, encw_e)
    s0, s1 = _sc_scatter(e, src, z128)
    c0, c1 = _sc_count(src, ones128, z128)
    d0, d1 = _sc_count(dst, ones128, z128)
    s, c, d = (s0, s1), (c0, c1), (d0, d1)
    x, xa, xb = _call_node_first(node_feat, encw_n, s, c, proj_w[0])

    for i in range(4):
        ga, gb = _sc_gather(xa, xb, src, dst)
        e_new, e = _call_edge_step(e, ga, gb, edge_w[i])
        q0, q1 = _sc_scatter(e_new, dst, z128)
        q = (q0, q1)
        if i < 3:
            x, xa, xb = _call_node_step(x, q, d, node_w[i], proj_w[i + 1])
        else:
            out = _call_node_last(x, q, d, node_w[i], decw)
    return out
